# Initial kernel scaffold; baseline (speedup 1.0000x reference)
import jax
import jax.numpy as jnp
from jax.experimental import pallas as pl


def kernel(memory, source_nodes, timestamps, time_diffs, W, b):
    raise NotImplementedError("write your pallas kernel here")


# speedup vs baseline: 1.0211x; 1.0211x over previous
import functools

import jax
import jax.numpy as jnp
from jax import lax
from jax.experimental import pallas as pl
from jax.experimental.pallas import tpu as pltpu
from jax.experimental.pallas import tpu_sc as plsc

M = 1000000
D = 128
B = 16384
NC = 2
NS = 16
L = 16
NW = NC * NS
BPW = B // NW
GCHUNK = 128
NG = BPW // GCHUNK


def _make_sc_kernel():
    mesh = plsc.VectorSubcoreMesh(core_axis_name="c", subcore_axis_name="s")

    @functools.partial(
        pl.kernel,
        mesh=mesh,
        out_type=jax.ShapeDtypeStruct((B, D), jnp.float32),
        compiler_params=pltpu.CompilerParams(needs_layout_passes=False),
        scratch_types=[
            pltpu.VMEM((NG, GCHUNK), jnp.int32),
            pltpu.VMEM((BPW,), jnp.float32),
            pltpu.VMEM((D,), jnp.float32),
            pltpu.VMEM((D,), jnp.float32),
            pltpu.VMEM((BPW, D), jnp.float32),
            pltpu.SemaphoreType.DMA,
        ],
    )
    def sc_kernel(mem_hbm, idx_hbm, td_hbm, w_hbm, b_hbm, out_hbm,
                  idx_v, td_v, w_v, b_v, rows_v, sem):
        wid = lax.axis_index("s") * NC + lax.axis_index("c")
        base = wid * BPW

        pltpu.sync_copy(idx_hbm.at[wid], idx_v)
        pltpu.sync_copy(td_hbm.at[pl.ds(base, BPW)], td_v)
        pltpu.sync_copy(w_hbm, w_v)
        pltpu.sync_copy(b_hbm, b_v)

        copies = []
        for g in range(NG):
            copies.append(pltpu.async_copy(
                mem_hbm.at[idx_v.at[g]],
                rows_v.at[pl.ds(g * GCHUNK, GCHUNK)],
                sem,
            ))
        for c in copies:
            c.wait()

        w_chunks = [w_v[pl.ds(c * L, L)] for c in range(D // L)]
        b_chunks = [b_v[pl.ds(c * L, L)] + 1.0 for c in range(D // L)]

        def row_body(i, carry):
            tdv = plsc.load_gather(td_v, [jnp.full((L,), i, jnp.int32)])
            for c in range(D // L):
                sl = pl.ds(c * L, L)
                rows_v[i, sl] = rows_v[i, sl] * (tdv * w_chunks[c] + b_chunks[c])
            return carry

        lax.fori_loop(0, BPW, row_body, 0)

        pltpu.sync_copy(rows_v, out_hbm.at[pl.ds(base, BPW)])

    return sc_kernel


_sc_kernel = _make_sc_kernel()


def kernel(memory, source_nodes, timestamps, time_diffs, W, b):
    del timestamps
    idx = source_nodes.astype(jnp.int32).reshape(NW, NG, GCHUNK)
    w_flat = W.reshape(D)
    return _sc_kernel(memory, idx, time_diffs.astype(jnp.float32), w_flat, b)


# speedup vs baseline: 1.1218x; 1.0987x over previous
import functools

import jax
import jax.numpy as jnp
from jax import lax
from jax.experimental import pallas as pl
from jax.experimental.pallas import tpu as pltpu
from jax.experimental.pallas import tpu_sc as plsc

M = 1000000
D = 128
B = 16384
NC = 2
NS = 16
L = 16
NW = NC * NS
BPW = B // NW
GCHUNK = 128
NG = BPW // GCHUNK


def _make_sc_kernel():
    mesh = plsc.VectorSubcoreMesh(core_axis_name="c", subcore_axis_name="s")

    @functools.partial(
        pl.kernel,
        mesh=mesh,
        out_type=jax.ShapeDtypeStruct((B, D), jnp.float32),
        compiler_params=pltpu.CompilerParams(needs_layout_passes=False),
        scratch_types=[
            pltpu.VMEM((NG, GCHUNK), jnp.int32),
            pltpu.VMEM((BPW,), jnp.float32),
            pltpu.VMEM((D,), jnp.float32),
            pltpu.VMEM((D,), jnp.float32),
            pltpu.VMEM((BPW, D), jnp.float32),
            pltpu.SemaphoreType.DMA,
            pltpu.SemaphoreType.DMA,
            [pltpu.SemaphoreType.DMA] * NG,
            pltpu.SemaphoreType.DMA,
        ],
    )
    def sc_kernel(mem_hbm, idx_hbm, td_hbm, w_hbm, b_hbm, out_hbm,
                  idx_v, td_v, w_v, b_v, rows_v,
                  sem_idx, sem_stage, sems_g, sem_out):
        wid = lax.axis_index("s") * NC + lax.axis_index("c")
        base = wid * BPW

        c_idx = pltpu.async_copy(idx_hbm.at[wid], idx_v, sem_idx)
        c_td = pltpu.async_copy(td_hbm.at[pl.ds(base, BPW)], td_v, sem_stage)
        c_w = pltpu.async_copy(w_hbm, w_v, sem_stage)
        c_b = pltpu.async_copy(b_hbm, b_v, sem_stage)

        c_idx.wait()
        gathers = []
        for g in range(NG):
            gathers.append(pltpu.async_copy(
                mem_hbm.at[idx_v.at[g]],
                rows_v.at[pl.ds(g * GCHUNK, GCHUNK)],
                sems_g[g],
            ))
        c_td.wait()
        c_w.wait()
        c_b.wait()

        w_chunks = [w_v[pl.ds(c * L, L)] for c in range(D // L)]
        b_chunks = [b_v[pl.ds(c * L, L)] + 1.0 for c in range(D // L)]

        stores = []
        for g in range(NG):
            gathers[g].wait()
            off = g * GCHUNK

            def row_body(i, carry):
                tdv = plsc.load_gather(td_v, [jnp.full((L,), i, jnp.int32)])
                for c in range(D // L):
                    sl = pl.ds(c * L, L)
                    rows_v[i, sl] = rows_v[i, sl] * (tdv * w_chunks[c] + b_chunks[c])
                return carry

            lax.fori_loop(off, off + GCHUNK, row_body, 0)

            stores.append(pltpu.async_copy(
                rows_v.at[pl.ds(off, GCHUNK)],
                out_hbm.at[pl.ds(base + off, GCHUNK)],
                sem_out,
            ))
        for s in stores:
            s.wait()

    return sc_kernel


_sc_kernel = _make_sc_kernel()


def kernel(memory, source_nodes, timestamps, time_diffs, W, b):
    del timestamps
    idx = source_nodes.astype(jnp.int32).reshape(NW, NG, GCHUNK)
    w_flat = W.reshape(D)
    return _sc_kernel(memory, idx, time_diffs.astype(jnp.float32), w_flat, b)


# speedup vs baseline: 1.1220x; 1.0001x over previous
import functools

import jax
import jax.numpy as jnp
from jax import lax
from jax.experimental import pallas as pl
from jax.experimental.pallas import tpu as pltpu
from jax.experimental.pallas import tpu_sc as plsc

M = 1000000
D = 128
B = 16384
NC = 2
NS = 16
L = 16
NW = NC * NS
BPW = B // NW
GCHUNK = 64
NG = BPW // GCHUNK
RUNROLL = 2


def _make_sc_kernel():
    mesh = plsc.VectorSubcoreMesh(core_axis_name="c", subcore_axis_name="s")

    @functools.partial(
        pl.kernel,
        mesh=mesh,
        out_type=jax.ShapeDtypeStruct((B, D), jnp.float32),
        compiler_params=pltpu.CompilerParams(needs_layout_passes=False),
        scratch_types=[
            pltpu.VMEM((NG, GCHUNK), jnp.int32),
            pltpu.VMEM((BPW,), jnp.float32),
            pltpu.VMEM((D,), jnp.float32),
            pltpu.VMEM((D,), jnp.float32),
            pltpu.VMEM((BPW, D), jnp.float32),
            pltpu.SemaphoreType.DMA,
            pltpu.SemaphoreType.DMA,
            [pltpu.SemaphoreType.DMA] * NG,
            pltpu.SemaphoreType.DMA,
        ],
    )
    def sc_kernel(mem_hbm, idx_hbm, td_hbm, w_hbm, b_hbm, out_hbm,
                  idx_v, td_v, w_v, b_v, rows_v,
                  sem_idx, sem_stage, sems_g, sem_out):
        wid = lax.axis_index("s") * NC + lax.axis_index("c")
        base = wid * BPW

        c_idx = pltpu.async_copy(idx_hbm.at[wid], idx_v, sem_idx)
        c_td = pltpu.async_copy(td_hbm.at[pl.ds(base, BPW)], td_v, sem_stage)
        c_w = pltpu.async_copy(w_hbm, w_v, sem_stage)
        c_b = pltpu.async_copy(b_hbm, b_v, sem_stage)

        c_idx.wait()
        gathers = []
        for g in range(NG):
            gathers.append(pltpu.async_copy(
                mem_hbm.at[idx_v.at[g]],
                rows_v.at[pl.ds(g * GCHUNK, GCHUNK)],
                sems_g[g],
            ))
        c_td.wait()
        c_w.wait()
        c_b.wait()

        w_chunks = [w_v[pl.ds(c * L, L)] for c in range(D // L)]
        b_chunks = [b_v[pl.ds(c * L, L)] + 1.0 for c in range(D // L)]

        def scale_row(i):
            tdv = plsc.load_gather(td_v, [jnp.full((L,), i, jnp.int32)])
            for c in range(D // L):
                sl = pl.ds(c * L, L)
                rows_v[i, sl] = rows_v[i, sl] * (tdv * w_chunks[c] + b_chunks[c])

        stores = []
        for g in range(NG):
            gathers[g].wait()
            off = g * GCHUNK

            def row_body(k, carry):
                i = off + k * RUNROLL
                for r in range(RUNROLL):
                    scale_row(i + r)
                return carry

            lax.fori_loop(0, GCHUNK // RUNROLL, row_body, 0)

            stores.append(pltpu.async_copy(
                rows_v.at[pl.ds(off, GCHUNK)],
                out_hbm.at[pl.ds(base + off, GCHUNK)],
                sem_out,
            ))
        for s in stores:
            s.wait()

    return sc_kernel


_sc_kernel = _make_sc_kernel()


def kernel(memory, source_nodes, timestamps, time_diffs, W, b):
    del timestamps
    idx = source_nodes.astype(jnp.int32).reshape(NW, NG, GCHUNK)
    w_flat = W.reshape(D)
    return _sc_kernel(memory, idx, time_diffs.astype(jnp.float32), w_flat, b)
